# baseline (speedup 1.0000x reference)
import functools
import itertools

import jax
import jax.numpy as jnp
import numpy as np
from jax import lax
from jax.experimental import pallas as pl
from jax.experimental.pallas import tpu as pltpu
from jax.experimental.pallas import tpu_sc as plsc

_F = 26
_V = 100000
_D = 16
_A = 8
_B = 4096
_LANES = _F * _D
_NSLAB = _F // 2
_GROUPS = _F * _V // 8

_SLAB_OFFS = [(o, _F - o) for o in range(1, _NSLAB)] + [(_NSLAB, None)]


def _make_sc_gather():
    info = plsc.get_sparse_core_info()
    nw = info.num_cores * info.num_subcores
    nc = info.num_cores
    per_w = _F * _B // nw
    n_sub = per_w // 128
    bw = _B // nw
    owords = bw * _LANES
    mesh = plsc.VectorSubcoreMesh(core_axis_name="c", subcore_axis_name="s")

    @functools.partial(
        pl.kernel,
        mesh=mesh,
        out_type=jax.ShapeDtypeStruct((nw, owords), jnp.float32),
        compiler_params=pltpu.CompilerParams(use_tc_tiling_on_sc=True,
                                             needs_layout_passes=False),
        scratch_types=[
            pltpu.VMEM((per_w,), jnp.int32),
            pltpu.VMEM((per_w,), jnp.int32),
            pltpu.VMEM((128, 128), jnp.float32),
            pltpu.VMEM((owords,), jnp.float32),
            pltpu.SemaphoreType.DMA,
        ],
    )
    def gather_k(ctab_hbm, qid_hbm, oid_hbm, out_hbm,
                 qid_v, oid_v, tiles_v, buf_v, sem):
        wid = lax.axis_index("s") * nc + lax.axis_index("c")
        pltpu.sync_copy(qid_hbm.at[wid], qid_v)
        pltpu.sync_copy(oid_hbm.at[wid], oid_v)
        iota = lax.iota(jnp.int32, 16)

        for j in range(n_sub):
            pltpu.async_copy(
                ctab_hbm.at[qid_v.at[pl.ds(j * 128, 128)]], tiles_v, sem
            ).wait()

            def extract(t, _, j=j):
                pos = j * 128 + t
                o16 = plsc.load_gather(
                    oid_v, [jnp.full((16,), pos, jnp.int32)])
                rows = jnp.full((16,), t, jnp.int32)
                cols = o16 * _D + iota
                vals = plsc.load_gather(tiles_v, [rows, cols])
                b_l = pos // _F
                f = pos - b_l * _F
                dst = b_l * _LANES + f * _D + iota
                plsc.store_scatter(buf_v, [dst], vals)
                return ()

            lax.fori_loop(0, 128, extract, ())

        pltpu.sync_copy(buf_v, out_hbm.at[wid])

    return gather_k, nw


def _attn_body(ep_ref, mh_ref, bat_ref, mv_ref, bv_ref, mt_ref, wo_ref,
               bo_ref, out_ref):
    ep = ep_ref[...]
    bt = ep.shape[0]
    mh = mh_ref[...]
    bat = bat_ref[...]
    mv = mv_ref[...]
    bv = bv_ref[0, 0]
    mt = mt_ref[...]
    wo = wo_ref[...]
    bo = bo_ref[0, 0]

    slabs = []
    for o1, o2 in _SLAB_OFFS:
        w1 = (_F - o1) * _D
        p1 = ep[:, :w1] * ep[:, o1 * _D:]
        if o2 is None:
            p1 = jnp.pad(p1, ((0, 0), (0, _LANES - w1)))
        else:
            w2 = (_F - o2) * _D
            p2 = ep[:, :w2] * ep[:, o2 * _D:]
            p1 = jnp.concatenate([p1, p2], axis=1)
        slabs.append(p1)
    bi = jnp.stack(slabs, axis=0)
    bi2 = bi.reshape(_NSLAB * bt, _LANES)

    h = jnp.maximum(
        jnp.dot(bi2, mh, preferred_element_type=jnp.float32) + bat, 0.0)
    lexp = (jnp.dot(h, mv, preferred_element_type=jnp.float32)
            + bv).reshape(_NSLAB, bt, _LANES)

    lane = lax.broadcasted_iota(jnp.int32, (1, 1, _LANES), 2)
    slab = lax.broadcasted_iota(jnp.int32, (_NSLAB, 1, 1), 0)
    valid = (slab < _NSLAB - 1) | (lane < (_F - _NSLAB) * _D)
    lm = jnp.where(valid, lexp, -1e30)

    m = jnp.max(jnp.max(lm, axis=0), axis=-1, keepdims=True)
    ex = jnp.exp(lm - m[None])
    s = jnp.sum(jnp.sum(ex, axis=0), axis=-1, keepdims=True) * (1.0 / _D)
    score = ex / s[None]

    weighted = (bi * score).reshape(_NSLAB * bt, _LANES)
    xs = jnp.dot(weighted, mt,
                 preferred_element_type=jnp.float32)
    x = jnp.sum(xs.reshape(_NSLAB, bt, _D), axis=0)

    z = jnp.sum(x * wo, axis=1, keepdims=True) + bo
    out_ref[...] = jax.nn.sigmoid(z)


def _attn_call(ep, mh, bat, mv, bv, mt, wo, bo, btile):
    grid = (_B // btile,)
    return pl.pallas_call(
        _attn_body,
        grid=grid,
        in_specs=[
            pl.BlockSpec((btile, _LANES), lambda i: (i, 0)),
            pl.BlockSpec((_LANES, _F * _A), lambda i: (0, 0)),
            pl.BlockSpec((1, _F * _A), lambda i: (0, 0)),
            pl.BlockSpec((_F * _A, _LANES), lambda i: (0, 0)),
            pl.BlockSpec((1, 1), lambda i: (0, 0)),
            pl.BlockSpec((_LANES, _D), lambda i: (0, 0)),
            pl.BlockSpec((1, _D), lambda i: (0, 0)),
            pl.BlockSpec((1, 1), lambda i: (0, 0)),
        ],
        out_specs=pl.BlockSpec((btile, 1), lambda i: (i, 0)),
        out_shape=jax.ShapeDtypeStruct((_B, 1), jnp.float32),
    )(ep, mh, bat, mv, bv, mt, wo, bo)


def kernel(inputs, tables, W_att, b_att, v_att, b_v, W_out, b_out):
    gather_k, nw = _make_sc_gather()
    ctab = tables.reshape(_GROUPS, 8 * _D)
    offs = jnp.arange(_F, dtype=jnp.int32) * _V
    flat_idx = inputs.astype(jnp.int32) + offs[None, :]
    qid = (flat_idx // 8).reshape(nw, _F * _B // nw)
    oid = (flat_idx % 8).reshape(nw, _F * _B // nw)
    emb = gather_k(ctab, qid, oid).reshape(_B, _LANES)

    eye_f = jnp.eye(_F, dtype=jnp.float32)
    mh = jnp.kron(eye_f, W_att)
    bat = jnp.tile(b_att, _F).reshape(1, _F * _A)
    mv = jnp.kron(eye_f, v_att * jnp.ones((1, _D)))
    mt = jnp.kron(jnp.ones((_F, 1), jnp.float32),
                  jnp.eye(_D, dtype=jnp.float32))

    out = _attn_call(
        emb, mh, bat, mv,
        b_v.reshape(1, 1),
        mt,
        W_out.reshape(1, _D),
        b_out.reshape(1, 1),
        btile=256,
    )
    return out


# speedup vs baseline: 2.4461x; 2.4461x over previous
import functools
import itertools

import jax
import jax.numpy as jnp
import numpy as np
from jax import lax
from jax.experimental import pallas as pl
from jax.experimental.pallas import tpu as pltpu
from jax.experimental.pallas import tpu_sc as plsc

_F = 26
_V = 100000
_D = 16
_A = 8
_B = 4096
_LANES = _F * _D
_NSLAB = _F // 2
_GROUPS = _F * _V // 8

_SLAB_OFFS = [(o, _F - o) for o in range(1, _NSLAB)] + [(_NSLAB, None)]


def _make_sc_gather():
    info = plsc.get_sparse_core_info()
    nw = info.num_cores * info.num_subcores
    nc = info.num_cores
    bw = _B // nw
    owords = bw * _LANES
    mesh = plsc.VectorSubcoreMesh(core_axis_name="c", subcore_axis_name="s")

    @functools.partial(
        pl.kernel,
        mesh=mesh,
        out_type=jax.ShapeDtypeStruct((nw, owords), jnp.float32),
        compiler_params=pltpu.CompilerParams(use_tc_tiling_on_sc=False),
        scratch_types=[
            pltpu.VMEM((owords,), jnp.int32),
            pltpu.VMEM((owords,), jnp.float32),
            pltpu.SemaphoreType.DMA,
        ],
    )
    def gather_k(tab_hbm, eid_hbm, out_hbm, eid_v, buf_v, sem):
        wid = lax.axis_index("s") * nc + lax.axis_index("c")
        pltpu.sync_copy(eid_hbm.at[wid], eid_v)
        pltpu.async_copy(tab_hbm.at[eid_v], buf_v, sem).wait()
        pltpu.sync_copy(buf_v, out_hbm.at[wid])

    return gather_k, nw


def _attn_body(ep_ref, mh_ref, bat_ref, mv_ref, bv_ref, mt_ref, wo_ref,
               bo_ref, out_ref):
    ep = ep_ref[...]
    bt = ep.shape[0]
    mh = mh_ref[...]
    bat = bat_ref[...]
    mv = mv_ref[...]
    bv = bv_ref[0, 0]
    mt = mt_ref[...]
    wo = wo_ref[...]
    bo = bo_ref[0, 0]

    slabs = []
    for o1, o2 in _SLAB_OFFS:
        w1 = (_F - o1) * _D
        p1 = ep[:, :w1] * ep[:, o1 * _D:]
        if o2 is None:
            p1 = jnp.pad(p1, ((0, 0), (0, _LANES - w1)))
        else:
            w2 = (_F - o2) * _D
            p2 = ep[:, :w2] * ep[:, o2 * _D:]
            p1 = jnp.concatenate([p1, p2], axis=1)
        slabs.append(p1)
    bi = jnp.stack(slabs, axis=0)
    bi2 = bi.reshape(_NSLAB * bt, _LANES)

    h = jnp.maximum(
        jnp.dot(bi2, mh, preferred_element_type=jnp.float32) + bat, 0.0)
    lexp = (jnp.dot(h, mv, preferred_element_type=jnp.float32)
            + bv).reshape(_NSLAB, bt, _LANES)

    lane = lax.broadcasted_iota(jnp.int32, (1, 1, _LANES), 2)
    slab = lax.broadcasted_iota(jnp.int32, (_NSLAB, 1, 1), 0)
    valid = (slab < _NSLAB - 1) | (lane < (_F - _NSLAB) * _D)
    lm = jnp.where(valid, lexp, -1e30)

    m = jnp.max(jnp.max(lm, axis=0), axis=-1, keepdims=True)
    ex = jnp.exp(lm - m[None])
    s = jnp.sum(jnp.sum(ex, axis=0), axis=-1, keepdims=True) * (1.0 / _D)
    score = ex / s[None]

    weighted = (bi * score).reshape(_NSLAB * bt, _LANES)
    xs = jnp.dot(weighted, mt,
                 preferred_element_type=jnp.float32)
    x = jnp.sum(xs.reshape(_NSLAB, bt, _D), axis=0)

    z = jnp.sum(x * wo, axis=1, keepdims=True) + bo
    out_ref[...] = jax.nn.sigmoid(z)


def _attn_call(ep, mh, bat, mv, bv, mt, wo, bo, btile):
    grid = (_B // btile,)
    return pl.pallas_call(
        _attn_body,
        grid=grid,
        in_specs=[
            pl.BlockSpec((btile, _LANES), lambda i: (i, 0)),
            pl.BlockSpec((_LANES, _F * _A), lambda i: (0, 0)),
            pl.BlockSpec((1, _F * _A), lambda i: (0, 0)),
            pl.BlockSpec((_F * _A, _LANES), lambda i: (0, 0)),
            pl.BlockSpec((1, 1), lambda i: (0, 0)),
            pl.BlockSpec((_LANES, _D), lambda i: (0, 0)),
            pl.BlockSpec((1, _D), lambda i: (0, 0)),
            pl.BlockSpec((1, 1), lambda i: (0, 0)),
        ],
        out_specs=pl.BlockSpec((btile, 1), lambda i: (i, 0)),
        out_shape=jax.ShapeDtypeStruct((_B, 1), jnp.float32),
    )(ep, mh, bat, mv, bv, mt, wo, bo)


def kernel(inputs, tables, W_att, b_att, v_att, b_v, W_out, b_out):
    gather_k, nw = _make_sc_gather()
    tab_flat = jnp.transpose(tables, (0, 2, 1)).reshape(_F * _D * _V)
    ridx = inputs.astype(jnp.int32)
    gd = jnp.arange(_LANES, dtype=jnp.int32) * _V
    eid = (jnp.repeat(ridx, _D, axis=1) + gd[None, :]).reshape(
        nw, (_B // nw) * _LANES)
    emb = gather_k(tab_flat, eid).reshape(_B, _LANES)

    eye_f = jnp.eye(_F, dtype=jnp.float32)
    mh = jnp.kron(eye_f, W_att)
    bat = jnp.tile(b_att, _F).reshape(1, _F * _A)
    mv = jnp.kron(eye_f, v_att * jnp.ones((1, _D)))
    mt = jnp.kron(jnp.ones((_F, 1), jnp.float32),
                  jnp.eye(_D, dtype=jnp.float32))

    out = _attn_call(
        emb, mh, bat, mv,
        b_v.reshape(1, 1),
        mt,
        W_out.reshape(1, _D),
        b_out.reshape(1, 1),
        btile=256,
    )
    return out


# speedup vs baseline: 2.7979x; 1.1439x over previous
import functools
import itertools

import jax
import jax.numpy as jnp
import numpy as np
from jax import lax
from jax.experimental import pallas as pl
from jax.experimental.pallas import tpu as pltpu
from jax.experimental.pallas import tpu_sc as plsc

_F = 26
_V = 100000
_D = 16
_A = 8
_B = 4096
_LANES = _F * _D
_NSLAB = _F // 2
_GROUPS = _F * _V // 8

_SLAB_OFFS = [(o, _F - o) for o in range(1, _NSLAB)] + [(_NSLAB, None)]


def _make_sc_gather():
    info = plsc.get_sparse_core_info()
    nw = info.num_cores * info.num_subcores
    nc = info.num_cores
    bw = _B // nw
    owords = bw * _LANES
    mesh = plsc.VectorSubcoreMesh(core_axis_name="c", subcore_axis_name="s")

    @functools.partial(
        pl.kernel,
        mesh=mesh,
        out_type=jax.ShapeDtypeStruct((nw, owords), jnp.float32),
        compiler_params=pltpu.CompilerParams(use_tc_tiling_on_sc=False,
                                             needs_layout_passes=False),
        scratch_types=[
            pltpu.VMEM((bw * _F,), jnp.int32),
            pltpu.VMEM((owords,), jnp.int32),
            pltpu.VMEM((owords,), jnp.float32),
            pltpu.SemaphoreType.DMA,
        ],
    )
    def gather_k(tab_hbm, rid_hbm, out_hbm, rid_v, eid_v, buf_v, sem):
        wid = lax.axis_index("s") * nc + lax.axis_index("c")
        pltpu.sync_copy(rid_hbm.at[wid], rid_v)
        iota = lax.iota(jnp.int32, 16)
        iv = iota * _V

        def build(p, _):
            r16 = plsc.load_gather(rid_v, [jnp.full((16,), p, jnp.int32)])
            f = lax.rem(p, _F)
            vals = r16 + f * (_D * _V) + iv
            plsc.store_scatter(eid_v, [p * _D + iota], vals)
            return ()

        lax.fori_loop(0, bw * _F, build, ())
        pltpu.async_copy(tab_hbm.at[eid_v], buf_v, sem).wait()
        pltpu.sync_copy(buf_v, out_hbm.at[wid])

    return gather_k, nw


def _attn_body(ep_ref, mh_ref, bat_ref, mv_ref, bv_ref, mt_ref, wo_ref,
               bo_ref, out_ref):
    ep = ep_ref[...]
    bt = ep.shape[0]
    mh = mh_ref[...]
    bat = bat_ref[...]
    mv = mv_ref[...]
    bv = bv_ref[0, 0]
    mt = mt_ref[...]
    wo = wo_ref[...]
    bo = bo_ref[0, 0]

    slabs = []
    for o1, o2 in _SLAB_OFFS:
        w1 = (_F - o1) * _D
        p1 = ep[:, :w1] * ep[:, o1 * _D:]
        if o2 is None:
            p1 = jnp.pad(p1, ((0, 0), (0, _LANES - w1)))
        else:
            w2 = (_F - o2) * _D
            p2 = ep[:, :w2] * ep[:, o2 * _D:]
            p1 = jnp.concatenate([p1, p2], axis=1)
        slabs.append(p1)
    bi = jnp.stack(slabs, axis=0)
    bi2 = bi.reshape(_NSLAB * bt, _LANES)

    h = jnp.maximum(
        jnp.dot(bi2, mh, preferred_element_type=jnp.float32) + bat, 0.0)
    lexp = (jnp.dot(h, mv, preferred_element_type=jnp.float32)
            + bv).reshape(_NSLAB, bt, _LANES)

    lane = lax.broadcasted_iota(jnp.int32, (1, 1, _LANES), 2)
    slab = lax.broadcasted_iota(jnp.int32, (_NSLAB, 1, 1), 0)
    valid = (slab < _NSLAB - 1) | (lane < (_F - _NSLAB) * _D)
    lm = jnp.where(valid, lexp, -1e30)

    m = jnp.max(jnp.max(lm, axis=0), axis=-1, keepdims=True)
    ex = jnp.exp(lm - m[None])
    s = jnp.sum(jnp.sum(ex, axis=0), axis=-1, keepdims=True) * (1.0 / _D)
    score = ex / s[None]

    weighted = (bi * score).reshape(_NSLAB * bt, _LANES)
    xs = jnp.dot(weighted, mt,
                 preferred_element_type=jnp.float32)
    x = jnp.sum(xs.reshape(_NSLAB, bt, _D), axis=0)

    z = jnp.sum(x * wo, axis=1, keepdims=True) + bo
    out_ref[...] = jax.nn.sigmoid(z)


def _attn_call(ep, mh, bat, mv, bv, mt, wo, bo, btile):
    grid = (_B // btile,)
    return pl.pallas_call(
        _attn_body,
        grid=grid,
        in_specs=[
            pl.BlockSpec((btile, _LANES), lambda i: (i, 0)),
            pl.BlockSpec((_LANES, _F * _A), lambda i: (0, 0)),
            pl.BlockSpec((1, _F * _A), lambda i: (0, 0)),
            pl.BlockSpec((_F * _A, _LANES), lambda i: (0, 0)),
            pl.BlockSpec((1, 1), lambda i: (0, 0)),
            pl.BlockSpec((_LANES, _D), lambda i: (0, 0)),
            pl.BlockSpec((1, _D), lambda i: (0, 0)),
            pl.BlockSpec((1, 1), lambda i: (0, 0)),
        ],
        out_specs=pl.BlockSpec((btile, 1), lambda i: (i, 0)),
        out_shape=jax.ShapeDtypeStruct((_B, 1), jnp.float32),
    )(ep, mh, bat, mv, bv, mt, wo, bo)


def kernel(inputs, tables, W_att, b_att, v_att, b_v, W_out, b_out):
    gather_k, nw = _make_sc_gather()
    tab_flat = jnp.transpose(tables, (0, 2, 1)).reshape(_F * _D * _V)
    rid = inputs.astype(jnp.int32).reshape(nw, (_B // nw) * _F)
    emb = gather_k(tab_flat, rid).reshape(_B, _LANES)

    eye_f = jnp.eye(_F, dtype=jnp.float32)
    mh = jnp.kron(eye_f, W_att)
    bat = jnp.tile(b_att, _F).reshape(1, _F * _A)
    mv = jnp.kron(eye_f, v_att * jnp.ones((1, _D)))
    mt = jnp.kron(jnp.ones((_F, 1), jnp.float32),
                  jnp.eye(_D, dtype=jnp.float32))

    out = _attn_call(
        emb, mh, bat, mv,
        b_v.reshape(1, 1),
        mt,
        W_out.reshape(1, _D),
        b_out.reshape(1, 1),
        btile=256,
    )
    return out


# speedup vs baseline: 2.8068x; 1.0032x over previous
import functools
import itertools

import jax
import jax.numpy as jnp
import numpy as np
from jax import lax
from jax.experimental import pallas as pl
from jax.experimental.pallas import tpu as pltpu
from jax.experimental.pallas import tpu_sc as plsc

_F = 26
_V = 100000
_D = 16
_A = 8
_B = 4096
_LANES = _F * _D
_NSLAB = _F // 2
_GROUPS = _F * _V // 8

_SLAB_OFFS = [(o, _F - o) for o in range(1, _NSLAB)] + [(_NSLAB, None)]


def _make_sc_gather():
    info = plsc.get_sparse_core_info()
    nw = info.num_cores * info.num_subcores
    nc = info.num_cores
    bw = _B // nw
    owords = bw * _LANES
    mesh = plsc.VectorSubcoreMesh(core_axis_name="c", subcore_axis_name="s")

    @functools.partial(
        pl.kernel,
        mesh=mesh,
        out_type=jax.ShapeDtypeStruct((nw, owords), jnp.float32),
        compiler_params=pltpu.CompilerParams(use_tc_tiling_on_sc=False,
                                             needs_layout_passes=False),
        scratch_types=[
            pltpu.VMEM((bw * _F,), jnp.int32),
            pltpu.VMEM((owords,), jnp.int32),
            pltpu.VMEM((owords,), jnp.float32),
            pltpu.SemaphoreType.DMA,
        ],
    )
    def gather_k(tab_hbm, rid_hbm, out_hbm, rid_v, eid_v, buf_v, sem):
        wid = lax.axis_index("s") * nc + lax.axis_index("c")
        pltpu.sync_copy(rid_hbm.at[wid], rid_v)
        iota = lax.iota(jnp.int32, 16)
        iv = iota * _V

        def build(p, _):
            r16 = plsc.load_gather(rid_v, [jnp.full((16,), p, jnp.int32)])
            f = lax.rem(p, _F)
            vals = r16 + f * (_D * _V) + iv
            plsc.store_scatter(eid_v, [p * _D + iota], vals)
            return ()

        lax.fori_loop(0, bw * _F, build, ())
        pltpu.async_copy(tab_hbm.at[eid_v], buf_v, sem).wait()
        pltpu.sync_copy(buf_v, out_hbm.at[wid])

    return gather_k, nw


def _attn_body(ep_ref, mh_ref, bat_ref, mv_ref, bv_ref, mt_ref, wo_ref,
               bo_ref, out_ref):
    ep = ep_ref[...]
    bt = ep.shape[0]
    mh = mh_ref[...]
    bat = bat_ref[...]
    mv = mv_ref[...]
    bv = bv_ref[0, 0]
    mt = mt_ref[...]
    wo = wo_ref[...]
    bo = bo_ref[0, 0]

    slabs = []
    for o1, o2 in _SLAB_OFFS:
        w1 = (_F - o1) * _D
        p1 = ep[:, :w1] * ep[:, o1 * _D:]
        if o2 is None:
            p1 = jnp.pad(p1, ((0, 0), (0, _LANES - w1)))
        else:
            w2 = (_F - o2) * _D
            p2 = ep[:, :w2] * ep[:, o2 * _D:]
            p1 = jnp.concatenate([p1, p2], axis=1)
        slabs.append(p1)
    bi = jnp.stack(slabs, axis=0)
    bi2 = bi.reshape(_NSLAB * bt, _LANES)

    h = jnp.maximum(
        jnp.dot(bi2, mh, preferred_element_type=jnp.float32) + bat, 0.0)
    lexp = (jnp.dot(h, mv, preferred_element_type=jnp.float32)
            + bv).reshape(_NSLAB, bt, _LANES)

    lane = lax.broadcasted_iota(jnp.int32, (1, 1, _LANES), 2)
    slab = lax.broadcasted_iota(jnp.int32, (_NSLAB, 1, 1), 0)
    valid = (slab < _NSLAB - 1) | (lane < (_F - _NSLAB) * _D)
    lm = jnp.where(valid, lexp, -1e30)

    m = jnp.max(jnp.max(lm, axis=0), axis=-1, keepdims=True)
    ex = jnp.exp(lm - m[None])
    s = jnp.sum(jnp.sum(ex, axis=0), axis=-1, keepdims=True) * (1.0 / _D)
    score = ex / s[None]

    weighted = (bi * score).reshape(_NSLAB * bt, _LANES)
    xs = jnp.dot(weighted, mt,
                 preferred_element_type=jnp.float32)
    x = jnp.sum(xs.reshape(_NSLAB, bt, _D), axis=0)

    z = jnp.sum(x * wo, axis=1, keepdims=True) + bo
    out_ref[...] = jax.nn.sigmoid(z)


def _attn_call(ep, mh, bat, mv, bv, mt, wo, bo, btile):
    grid = (_B // btile,)
    return pl.pallas_call(
        _attn_body,
        grid=grid,
        in_specs=[
            pl.BlockSpec((btile, _LANES), lambda i: (i, 0)),
            pl.BlockSpec((_LANES, _F * _A), lambda i: (0, 0)),
            pl.BlockSpec((1, _F * _A), lambda i: (0, 0)),
            pl.BlockSpec((_F * _A, _LANES), lambda i: (0, 0)),
            pl.BlockSpec((1, 1), lambda i: (0, 0)),
            pl.BlockSpec((_LANES, _D), lambda i: (0, 0)),
            pl.BlockSpec((1, _D), lambda i: (0, 0)),
            pl.BlockSpec((1, 1), lambda i: (0, 0)),
        ],
        out_specs=pl.BlockSpec((btile, 1), lambda i: (i, 0)),
        out_shape=jax.ShapeDtypeStruct((_B, 1), jnp.float32),
    )(ep, mh, bat, mv, bv, mt, wo, bo)


def kernel(inputs, tables, W_att, b_att, v_att, b_v, W_out, b_out):
    gather_k, nw = _make_sc_gather()
    tab_flat = jnp.transpose(tables, (0, 2, 1)).reshape(_F * _D * _V)
    rid = inputs.astype(jnp.int32).reshape(nw, (_B // nw) * _F)
    emb = gather_k(tab_flat, rid).reshape(_B, _LANES)

    eye_f = jnp.eye(_F, dtype=jnp.float32)
    mh = jnp.kron(eye_f, W_att)
    bat = jnp.tile(b_att, _F).reshape(1, _F * _A)
    mv = jnp.kron(eye_f, v_att * jnp.ones((1, _D)))
    mt = jnp.kron(jnp.ones((_F, 1), jnp.float32),
                  jnp.eye(_D, dtype=jnp.float32))

    out = _attn_call(
        emb, mh, bat, mv,
        b_v.reshape(1, 1),
        mt,
        W_out.reshape(1, _D),
        b_out.reshape(1, 1),
        btile=512,
    )
    return out


# speedup vs baseline: 3.0204x; 1.0761x over previous
import functools
import itertools

import jax
import jax.numpy as jnp
import numpy as np
from jax import lax
from jax.experimental import pallas as pl
from jax.experimental.pallas import tpu as pltpu
from jax.experimental.pallas import tpu_sc as plsc

_F = 26
_V = 100000
_D = 16
_A = 8
_B = 4096
_LANES = _F * _D
_NSLAB = _F // 2
_GROUPS = _F * _V // 8

_SLAB_OFFS = [(o, _F - o) for o in range(1, _NSLAB)] + [(_NSLAB, None)]


def _make_sc_gather(nb):
    info = plsc.get_sparse_core_info()
    nw = info.num_cores * info.num_subcores
    nc = info.num_cores
    bw = nb // nw
    owords = bw * _LANES
    mesh = plsc.VectorSubcoreMesh(core_axis_name="c", subcore_axis_name="s")

    @functools.partial(
        pl.kernel,
        mesh=mesh,
        out_type=jax.ShapeDtypeStruct((nw, owords), jnp.float32),
        compiler_params=pltpu.CompilerParams(use_tc_tiling_on_sc=False,
                                             needs_layout_passes=False),
        scratch_types=[
            pltpu.VMEM((bw * _F,), jnp.int32),
            pltpu.VMEM((owords,), jnp.int32),
            pltpu.VMEM((owords,), jnp.float32),
            pltpu.SemaphoreType.DMA,
        ],
    )
    def gather_k(tab_hbm, rid_hbm, out_hbm, rid_v, eid_v, buf_v, sem):
        wid = lax.axis_index("s") * nc + lax.axis_index("c")
        pltpu.sync_copy(rid_hbm.at[wid], rid_v)
        iota = lax.iota(jnp.int32, 16)
        iv = iota * _V

        def build(p, _):
            r16 = plsc.load_gather(rid_v, [jnp.full((16,), p, jnp.int32)])
            f = lax.rem(p, _F)
            vals = r16 + f * (_D * _V) + iv
            plsc.store_scatter(eid_v, [p * _D + iota], vals)
            return ()

        lax.fori_loop(0, bw * _F, build, ())
        pltpu.async_copy(tab_hbm.at[eid_v], buf_v, sem).wait()
        pltpu.sync_copy(buf_v, out_hbm.at[wid])

    return gather_k, nw


def _attn_body(ep_ref, mh_ref, bat_ref, mv_ref, bv_ref, mt_ref, wo_ref,
               bo_ref, out_ref):
    ep = ep_ref[...]
    bt = ep.shape[0]
    mh = mh_ref[...]
    bat = bat_ref[...]
    mv = mv_ref[...]
    bv = bv_ref[0, 0]
    mt = mt_ref[...]
    wo = wo_ref[...]
    bo = bo_ref[0, 0]

    slabs = []
    for o1, o2 in _SLAB_OFFS:
        w1 = (_F - o1) * _D
        p1 = ep[:, :w1] * ep[:, o1 * _D:]
        if o2 is None:
            p1 = jnp.pad(p1, ((0, 0), (0, _LANES - w1)))
        else:
            w2 = (_F - o2) * _D
            p2 = ep[:, :w2] * ep[:, o2 * _D:]
            p1 = jnp.concatenate([p1, p2], axis=1)
        slabs.append(p1)
    bi = jnp.stack(slabs, axis=0)
    bi2 = bi.reshape(_NSLAB * bt, _LANES)

    h = jnp.maximum(
        jnp.dot(bi2, mh, preferred_element_type=jnp.float32) + bat, 0.0)
    lexp = (jnp.dot(h, mv, preferred_element_type=jnp.float32)
            + bv).reshape(_NSLAB, bt, _LANES)

    lane = lax.broadcasted_iota(jnp.int32, (1, 1, _LANES), 2)
    slab = lax.broadcasted_iota(jnp.int32, (_NSLAB, 1, 1), 0)
    valid = (slab < _NSLAB - 1) | (lane < (_F - _NSLAB) * _D)
    lm = jnp.where(valid, lexp, -1e30)

    m = jnp.max(jnp.max(lm, axis=0), axis=-1, keepdims=True)
    ex = jnp.exp(lm - m[None])
    s = jnp.sum(jnp.sum(ex, axis=0), axis=-1, keepdims=True) * (1.0 / _D)
    score = ex / s[None]

    weighted = (bi * score).reshape(_NSLAB * bt, _LANES)
    xs = jnp.dot(weighted, mt,
                 preferred_element_type=jnp.float32)
    x = jnp.sum(xs.reshape(_NSLAB, bt, _D), axis=0)

    z = jnp.sum(x * wo, axis=1, keepdims=True) + bo
    out_ref[...] = jax.nn.sigmoid(z)


def _attn_call(ep, mh, bat, mv, bv, mt, wo, bo, btile):
    nb = ep.shape[0]
    grid = (nb // btile,)
    return pl.pallas_call(
        _attn_body,
        grid=grid,
        in_specs=[
            pl.BlockSpec((btile, _LANES), lambda i: (i, 0)),
            pl.BlockSpec((_LANES, _F * _A), lambda i: (0, 0)),
            pl.BlockSpec((1, _F * _A), lambda i: (0, 0)),
            pl.BlockSpec((_F * _A, _LANES), lambda i: (0, 0)),
            pl.BlockSpec((1, 1), lambda i: (0, 0)),
            pl.BlockSpec((_LANES, _D), lambda i: (0, 0)),
            pl.BlockSpec((1, _D), lambda i: (0, 0)),
            pl.BlockSpec((1, 1), lambda i: (0, 0)),
        ],
        out_specs=pl.BlockSpec((btile, 1), lambda i: (i, 0)),
        out_shape=jax.ShapeDtypeStruct((nb, 1), jnp.float32),
    )(ep, mh, bat, mv, bv, mt, wo, bo)


def kernel(inputs, tables, W_att, b_att, v_att, b_v, W_out, b_out):
    nh = _B // 2
    gather_k, nw = _make_sc_gather(nh)
    tab_flat = jnp.transpose(tables, (0, 2, 1)).reshape(_F * _D * _V)
    rid = inputs.astype(jnp.int32).reshape(2, nw, (nh // nw) * _F)
    emb1 = gather_k(tab_flat, rid[0]).reshape(nh, _LANES)
    emb2 = gather_k(tab_flat, rid[1]).reshape(nh, _LANES)

    eye_f = jnp.eye(_F, dtype=jnp.float32)
    mh = jnp.kron(eye_f, W_att)
    bat = jnp.tile(b_att, _F).reshape(1, _F * _A)
    mv = jnp.kron(eye_f, v_att * jnp.ones((1, _D)))
    mt = jnp.kron(jnp.ones((_F, 1), jnp.float32),
                  jnp.eye(_D, dtype=jnp.float32))

    wargs = (mh, bat, mv, b_v.reshape(1, 1), mt, W_out.reshape(1, _D),
             b_out.reshape(1, 1))
    out1 = _attn_call(emb1, *wargs, btile=512)
    out2 = _attn_call(emb2, *wargs, btile=512)
    return jnp.concatenate([out1, out2], axis=0)
